# Initial kernel scaffold; baseline (speedup 1.0000x reference)
import jax
import jax.numpy as jnp
from jax.experimental import pallas as pl


def kernel(q, mu, W_ij, dir_ij, pairlist, W1, b1, W2, b2):
    raise NotImplementedError("write your pallas kernel here")


# speedup vs baseline: 6.9660x; 6.9660x over previous
import functools

import jax
import jax.numpy as jnp
from jax import lax
from jax.experimental import pallas as pl
from jax.experimental.pallas import tpu as pltpu
from jax.experimental.pallas import tpu_sc as plsc

N = 10000
E = 320000
F = 128
NSUB = 16
EPT = E // NSUB
B = 40
NCHUNK = EPT // B
RPT = (N // NSUB) // 8 * 8
TAIL = N - RPT * NSUB
NREG = F // 16


def _mlp_body(q_ref, w1_ref, b1_ref, w2_ref, b2_ref, x0_ref, x1_ref, x2_ref):
    h = jnp.dot(q_ref[...], w1_ref[...], preferred_element_type=jnp.float32)
    h = h + b1_ref[...]
    h = h * jax.nn.sigmoid(h)
    x = jnp.dot(h, w2_ref[...], preferred_element_type=jnp.float32) + b2_ref[...]
    x0_ref[...] = x[:, :F]
    x1_ref[...] = x[:, F:2 * F]
    x2_ref[...] = x[:, 2 * F:]


def _mlp(q2, W1, b1, W2, b2):
    BN = 1000
    grid = (N // BN,)
    out = pl.pallas_call(
        _mlp_body,
        grid=grid,
        in_specs=[
            pl.BlockSpec((BN, F), lambda i: (i, 0)),
            pl.BlockSpec((F, F), lambda i: (0, 0)),
            pl.BlockSpec((1, F), lambda i: (0, 0)),
            pl.BlockSpec((F, 3 * F), lambda i: (0, 0)),
            pl.BlockSpec((1, 3 * F), lambda i: (0, 0)),
        ],
        out_specs=[
            pl.BlockSpec((BN, F), lambda i: (i, 0)),
            pl.BlockSpec((BN, F), lambda i: (i, 0)),
            pl.BlockSpec((BN, F), lambda i: (i, 0)),
        ],
        out_shape=[jax.ShapeDtypeStruct((N, F), jnp.float32)] * 3,
    )(q2, W1, b1.reshape(1, F), W2, b2.reshape(1, 3 * F))
    return out


def _sc_edge_kernel(w_hbm, x0_hbm, x1_hbm, x2_hbm,
                    mu0_hbm, mu1_hbm, mu2_hbm,
                    d0_hbm, d1_hbm, d2_hbm,
                    q2_hbm, idxi_hbm, idxj_hbm,
                    outq_hbm, outm0_hbm, outm1_hbm, outm2_hbm,
                    acc, idxi_v, idxj_v, dir_v,
                    w1_v, w2_v, xa_v, xb_v, mu_v, sem):
    cid = lax.axis_index("c")
    sid = lax.axis_index("s")
    row0 = sid * RPT
    ebase = sid * EPT

    def init_acc(base_hbm):
        pltpu.sync_copy(base_hbm.at[pl.ds(row0, RPT), :],
                        acc.at[pl.ds(row0, RPT), :])

        @pl.when(sid == NSUB - 1)
        def _():
            pltpu.sync_copy(base_hbm.at[pl.ds(RPT * NSUB, TAIL), :],
                            acc.at[pl.ds(RPT * NSUB, TAIL), :])

        plsc.subcore_barrier()

    def writeout(out_hbm):
        plsc.subcore_barrier()
        pltpu.sync_copy(acc.at[pl.ds(row0, RPT), :],
                        out_hbm.at[pl.ds(row0, RPT), :])

        @pl.when(sid == NSUB - 1)
        def _():
            pltpu.sync_copy(acc.at[pl.ds(RPT * NSUB, TAIL), :],
                            out_hbm.at[pl.ds(RPT * NSUB, TAIL), :])

        plsc.subcore_barrier()

    def dq_pass():
        init_acc(q2_hbm)

        @pl.loop(0, NCHUNK)
        def _chunk(k):
            e0 = ebase + k * B
            pltpu.sync_copy(idxj_hbm.at[pl.ds(e0, B)], idxj_v)
            g1 = pltpu.async_copy(x0_hbm.at[idxj_v], xa_v, sem)
            pltpu.sync_copy(w_hbm.at[pl.ds(e0, B), pl.ds(0, F)], w1_v)
            pltpu.sync_copy(idxi_hbm.at[pl.ds(e0, B)], idxi_v)
            g1.wait()

            @pl.loop(0, B)
            def _edge(e):
                for r in range(NREG):
                    sl = pl.ds(r * 16, 16)
                    w1_v[e, sl] = w1_v[e, sl] * xa_v[e, sl]

            pltpu.sync_copy(w1_v, acc.at[idxi_v], add=True)

        writeout(outq_hbm)

    def dmu_pass(dcol_hbm, mu_hbm, out_hbm):
        init_acc(mu_hbm)

        @pl.loop(0, NCHUNK)
        def _chunk(k):
            e0 = ebase + k * B
            pltpu.sync_copy(idxj_hbm.at[pl.ds(e0, B)], idxj_v)
            g1 = pltpu.async_copy(x1_hbm.at[idxj_v], xa_v, sem)
            g2 = pltpu.async_copy(x2_hbm.at[idxj_v], xb_v, sem)
            g3 = pltpu.async_copy(mu_hbm.at[idxj_v], mu_v, sem)
            pltpu.sync_copy(w_hbm.at[pl.ds(e0, B), pl.ds(F, F)], w1_v)
            pltpu.sync_copy(w_hbm.at[pl.ds(e0, B), pl.ds(2 * F, F)], w2_v)
            pltpu.sync_copy(dcol_hbm.at[pl.ds(e0, B)], dir_v)
            pltpu.sync_copy(idxi_hbm.at[pl.ds(e0, B)], idxi_v)
            g1.wait()
            g2.wait()
            g3.wait()

            @pl.loop(0, B)
            def _edge(e):
                esplat = jnp.broadcast_to(e, (16,)).astype(jnp.int32)
                d = plsc.load_gather(dir_v, [esplat])
                for r in range(NREG):
                    sl = pl.ds(r * 16, 16)
                    w1_v[e, sl] = (w1_v[e, sl] * xa_v[e, sl] * d
                                   + w2_v[e, sl] * xb_v[e, sl] * mu_v[e, sl])

            pltpu.sync_copy(w1_v, acc.at[idxi_v], add=True)

        writeout(out_hbm)

    @pl.when(cid == 0)
    def _():
        dq_pass()
        dmu_pass(d0_hbm, mu0_hbm, outm0_hbm)

    @pl.when(cid == 1)
    def _():
        dmu_pass(d1_hbm, mu1_hbm, outm1_hbm)
        dmu_pass(d2_hbm, mu2_hbm, outm2_hbm)


_sc_edges = functools.partial(
    pl.kernel,
    out_type=[jax.ShapeDtypeStruct((N, F), jnp.float32)] * 4,
    mesh=plsc.VectorSubcoreMesh(core_axis_name="c", subcore_axis_name="s"),
    compiler_params=pltpu.CompilerParams(needs_layout_passes=False),
    scratch_types=[
        pltpu.VMEM_SHARED((N, F), jnp.float32),
        pltpu.VMEM((B,), jnp.int32),
        pltpu.VMEM((B,), jnp.int32),
        pltpu.VMEM((B,), jnp.float32),
        pltpu.VMEM((B, F), jnp.float32),
        pltpu.VMEM((B, F), jnp.float32),
        pltpu.VMEM((B, F), jnp.float32),
        pltpu.VMEM((B, F), jnp.float32),
        pltpu.VMEM((B, F), jnp.float32),
        pltpu.SemaphoreType.DMA,
    ],
)(_sc_edge_kernel)


def kernel(q, mu, W_ij, dir_ij, pairlist, W1, b1, W2, b2):
    q2 = q[:, 0, :]
    x0, x1, x2 = _mlp(q2, W1, b1, W2, b2)
    mu0, mu1, mu2 = mu[:, 0, :], mu[:, 1, :], mu[:, 2, :]
    d0, d1, d2 = dir_ij[:, 0], dir_ij[:, 1], dir_ij[:, 2]
    idx_i, idx_j = pairlist[0], pairlist[1]
    outq, m0, m1, m2 = _sc_edges(W_ij, x0, x1, x2, mu0, mu1, mu2,
                                 d0, d1, d2, q2, idx_i, idx_j)
    return outq[:, None, :], jnp.stack([m0, m1, m2], axis=1)


# speedup vs baseline: 9.4891x; 1.3622x over previous
import functools

import jax
import jax.numpy as jnp
from jax import lax
from jax.experimental import pallas as pl
from jax.experimental.pallas import tpu as pltpu
from jax.experimental.pallas import tpu_sc as plsc

N = 10000
E = 320000
F = 128
NSUB = 16
EPT = E // NSUB
B = 32
NCHUNK = EPT // B
RPT = (N // NSUB) // 8 * 8
TAIL = N - RPT * NSUB
NREG = F // 16


def _mlp_body(q_ref, w1_ref, b1_ref, w2_ref, b2_ref, x0_ref, x1_ref, x2_ref):
    h = jnp.dot(q_ref[...], w1_ref[...], preferred_element_type=jnp.float32)
    h = h + b1_ref[...]
    h = h * jax.nn.sigmoid(h)
    x = jnp.dot(h, w2_ref[...], preferred_element_type=jnp.float32) + b2_ref[...]
    x0_ref[...] = x[:, :F]
    x1_ref[...] = x[:, F:2 * F]
    x2_ref[...] = x[:, 2 * F:]


def _mlp(q2, W1, b1, W2, b2):
    BN = 1000
    grid = (N // BN,)
    out = pl.pallas_call(
        _mlp_body,
        grid=grid,
        in_specs=[
            pl.BlockSpec((BN, F), lambda i: (i, 0)),
            pl.BlockSpec((F, F), lambda i: (0, 0)),
            pl.BlockSpec((1, F), lambda i: (0, 0)),
            pl.BlockSpec((F, 3 * F), lambda i: (0, 0)),
            pl.BlockSpec((1, 3 * F), lambda i: (0, 0)),
        ],
        out_specs=[
            pl.BlockSpec((BN, F), lambda i: (i, 0)),
            pl.BlockSpec((BN, F), lambda i: (i, 0)),
            pl.BlockSpec((BN, F), lambda i: (i, 0)),
        ],
        out_shape=[jax.ShapeDtypeStruct((N, F), jnp.float32)] * 3,
    )(q2, W1, b1.reshape(1, F), W2, b2.reshape(1, 3 * F))
    return out


def _sc_edge_kernel(w_hbm, x0_hbm, x1_hbm, x2_hbm,
                    mu0_hbm, mu1_hbm, mu2_hbm,
                    d0_hbm, d1_hbm, d2_hbm,
                    q2_hbm, idxi_hbm, idxj_hbm,
                    outq_hbm, outm0_hbm, outm1_hbm, outm2_hbm,
                    acc, *scr):
    cid = lax.axis_index("c")
    sid = lax.axis_index("s")
    row0 = sid * RPT
    ebase = sid * EPT

    sets = (scr[0:9], scr[9:18])
    sem_load = scr[18:20]
    sem_scat = scr[20:22]

    def init_acc(base_hbm):
        pltpu.sync_copy(base_hbm.at[pl.ds(row0, RPT), :],
                        acc.at[pl.ds(row0, RPT), :])

        @pl.when(sid == NSUB - 1)
        def _():
            pltpu.sync_copy(base_hbm.at[pl.ds(RPT * NSUB, TAIL), :],
                            acc.at[pl.ds(RPT * NSUB, TAIL), :])

        plsc.subcore_barrier()

    def writeout(out_hbm):
        plsc.subcore_barrier()
        pltpu.sync_copy(acc.at[pl.ds(row0, RPT), :],
                        out_hbm.at[pl.ds(row0, RPT), :])

        @pl.when(sid == NSUB - 1)
        def _():
            pltpu.sync_copy(acc.at[pl.ds(RPT * NSUB, TAIL), :],
                            out_hbm.at[pl.ds(RPT * NSUB, TAIL), :])

        plsc.subcore_barrier()

    def run_pass(is_dq, x_a_hbm, x_b_hbm, mu_hbm, dcol_hbm, base_hbm, out_hbm):
        init_acc(base_hbm)
        wcol = 0 if is_dq else F

        def issue_loads(s, k):
            idxi_v, idxj_v, _, dir_v, w12_v, xa_v, xb_v, mu_v, _ = sets[s]
            e0 = ebase + k * B
            pltpu.sync_copy(idxj_hbm.at[pl.ds(e0, B)], idxj_v)
            pltpu.sync_copy(idxi_hbm.at[pl.ds(e0, B)], idxi_v)
            pltpu.async_copy(x_a_hbm.at[idxj_v], xa_v, sem_load[s])
            pltpu.async_copy(w_hbm.at[pl.ds(e0, B), pl.ds(wcol, 2 * F)],
                             w12_v, sem_load[s])
            if not is_dq:
                pltpu.async_copy(x_b_hbm.at[idxj_v], xb_v, sem_load[s])
                pltpu.async_copy(mu_hbm.at[idxj_v], mu_v, sem_load[s])
                pltpu.sync_copy(dcol_hbm.at[pl.ds(e0, B)], dir_v)

        def wait_loads(s):
            idxi_v, idxj_v, _, dir_v, w12_v, xa_v, xb_v, mu_v, _ = sets[s]
            pltpu.make_async_copy(x_a_hbm.at[idxj_v], xa_v, sem_load[s]).wait()
            pltpu.make_async_copy(w_hbm.at[pl.ds(0, B), pl.ds(wcol, 2 * F)],
                                  w12_v, sem_load[s]).wait()
            if not is_dq:
                pltpu.make_async_copy(x_b_hbm.at[idxj_v], xb_v,
                                      sem_load[s]).wait()
                pltpu.make_async_copy(mu_hbm.at[idxj_v], mu_v,
                                      sem_load[s]).wait()

        def compute(s):
            _, _, _, dir_v, w12_v, xa_v, xb_v, mu_v, out_v = sets[s]
            if is_dq:
                @pl.loop(0, B)
                def _edge(e):
                    for r in range(NREG):
                        sl = pl.ds(r * 16, 16)
                        out_v[e, sl] = w12_v[e, sl] * xa_v[e, sl]
            else:
                @pl.loop(0, B)
                def _edge(e):
                    esplat = jnp.broadcast_to(e, (16,)).astype(jnp.int32)
                    d = plsc.load_gather(dir_v, [esplat])
                    for r in range(NREG):
                        sl = pl.ds(r * 16, 16)
                        sl2 = pl.ds(F + r * 16, 16)
                        out_v[e, sl] = (w12_v[e, sl] * xa_v[e, sl] * d
                                        + w12_v[e, sl2] * xb_v[e, sl]
                                        * mu_v[e, sl])

        def issue_scatter(s):
            idxi_v, _, idxs_v, _, _, _, _, _, out_v = sets[s]
            for t in range(B // 16):
                sl = pl.ds(t * 16, 16)
                idxs_v[sl] = idxi_v[sl]
            pltpu.async_copy(out_v, acc.at[idxs_v], sem_scat[s], add=True)

        def wait_scatter(s):
            _, _, idxs_v, _, _, _, _, _, out_v = sets[s]
            pltpu.make_async_copy(out_v, acc.at[idxs_v], sem_scat[s]).wait()

        issue_loads(0, 0)
        issue_loads(1, 1)

        @pl.loop(0, (NCHUNK + 1) // 2)
        def _pair(m):
            for s in (0, 1):
                k = 2 * m + s

                @pl.when(k < NCHUNK)
                def _():
                    wait_loads(s)

                    @pl.when(m > 0)
                    def _():
                        wait_scatter(s)

                    compute(s)
                    issue_scatter(s)

                    @pl.when(k + 2 < NCHUNK)
                    def _():
                        issue_loads(s, k + 2)

        wait_scatter(0)
        wait_scatter(1)
        writeout(out_hbm)

    @pl.when(cid == 0)
    def _():
        run_pass(True, x0_hbm, x0_hbm, mu0_hbm, d0_hbm, q2_hbm, outq_hbm)
        run_pass(False, x1_hbm, x2_hbm, mu0_hbm, d0_hbm, mu0_hbm, outm0_hbm)

    @pl.when(cid == 1)
    def _():
        run_pass(False, x1_hbm, x2_hbm, mu1_hbm, d1_hbm, mu1_hbm, outm1_hbm)
        run_pass(False, x1_hbm, x2_hbm, mu2_hbm, d2_hbm, mu2_hbm, outm2_hbm)


def _set_scratch():
    return [
        pltpu.VMEM((B,), jnp.int32),
        pltpu.VMEM((B,), jnp.int32),
        pltpu.VMEM((B,), jnp.int32),
        pltpu.VMEM((B,), jnp.float32),
        pltpu.VMEM((B, 2 * F), jnp.float32),
        pltpu.VMEM((B, F), jnp.float32),
        pltpu.VMEM((B, F), jnp.float32),
        pltpu.VMEM((B, F), jnp.float32),
        pltpu.VMEM((B, F), jnp.float32),
    ]


_sc_edges = functools.partial(
    pl.kernel,
    out_type=[jax.ShapeDtypeStruct((N, F), jnp.float32)] * 4,
    mesh=plsc.VectorSubcoreMesh(core_axis_name="c", subcore_axis_name="s"),
    compiler_params=pltpu.CompilerParams(needs_layout_passes=False),
    scratch_types=[pltpu.VMEM_SHARED((N, F), jnp.float32)]
    + _set_scratch() + _set_scratch()
    + [pltpu.SemaphoreType.DMA] * 4,
)(_sc_edge_kernel)


def kernel(q, mu, W_ij, dir_ij, pairlist, W1, b1, W2, b2):
    q2 = q[:, 0, :]
    x0, x1, x2 = _mlp(q2, W1, b1, W2, b2)
    mu0, mu1, mu2 = mu[:, 0, :], mu[:, 1, :], mu[:, 2, :]
    d0, d1, d2 = dir_ij[:, 0], dir_ij[:, 1], dir_ij[:, 2]
    idx_i, idx_j = pairlist[0], pairlist[1]
    outq, m0, m1, m2 = _sc_edges(W_ij, x0, x1, x2, mu0, mu1, mu2,
                                 d0, d1, d2, q2, idx_i, idx_j)
    return outq[:, None, :], jnp.stack([m0, m1, m2], axis=1)


# speedup vs baseline: 10.4002x; 1.0960x over previous
import functools

import jax
import jax.numpy as jnp
from jax import lax
from jax.experimental import pallas as pl
from jax.experimental.pallas import tpu as pltpu
from jax.experimental.pallas import tpu_sc as plsc

N = 10000
E = 320000
F = 128
NSUB = 16
EPT = E // NSUB
B = 32
NCHUNK = EPT // B
RPT = (N // NSUB) // 8 * 8
TAIL = N - RPT * NSUB
NREG = F // 16


def _mlp_body(q_ref, w1_ref, b1_ref, w2_ref, b2_ref,
              mu0_ref, mu1_ref, mu2_ref,
              x0_ref, x1_ref, p0_ref, p1_ref, p2_ref):
    h = jnp.dot(q_ref[...], w1_ref[...], preferred_element_type=jnp.float32)
    h = h + b1_ref[...]
    h = h * jax.nn.sigmoid(h)
    x = jnp.dot(h, w2_ref[...], preferred_element_type=jnp.float32) + b2_ref[...]
    x0_ref[...] = x[:, :F]
    x1_ref[...] = x[:, F:2 * F]
    x2 = x[:, 2 * F:]
    p0_ref[...] = x2 * mu0_ref[...]
    p1_ref[...] = x2 * mu1_ref[...]
    p2_ref[...] = x2 * mu2_ref[...]


def _mlp(q2, W1, b1, W2, b2, mu0, mu1, mu2):
    BN = 1000
    grid = (N // BN,)
    blk = pl.BlockSpec((BN, F), lambda i: (i, 0))
    out = pl.pallas_call(
        _mlp_body,
        grid=grid,
        in_specs=[
            blk,
            pl.BlockSpec((F, F), lambda i: (0, 0)),
            pl.BlockSpec((1, F), lambda i: (0, 0)),
            pl.BlockSpec((F, 3 * F), lambda i: (0, 0)),
            pl.BlockSpec((1, 3 * F), lambda i: (0, 0)),
            blk, blk, blk,
        ],
        out_specs=[blk] * 5,
        out_shape=[jax.ShapeDtypeStruct((N, F), jnp.float32)] * 5,
    )(q2, W1, b1.reshape(1, F), W2, b2.reshape(1, 3 * F), mu0, mu1, mu2)
    return out


def _sc_edge_kernel(w_hbm, x0_hbm, x1_hbm, p0_hbm, p1_hbm, p2_hbm,
                    d0_hbm, d1_hbm, d2_hbm,
                    q2_hbm, mu0_hbm, mu1_hbm, mu2_hbm,
                    idxi_hbm, idxj_hbm,
                    outq_hbm, outm0_hbm, outm1_hbm, outm2_hbm,
                    acc, *scr):
    cid = lax.axis_index("c")
    sid = lax.axis_index("s")
    row0 = sid * RPT
    ebase = sid * EPT

    sets = (scr[0:7], scr[7:14])
    sem_load = scr[14:16]
    sem_scat = scr[16:18]

    def init_acc(base_hbm):
        pltpu.sync_copy(base_hbm.at[pl.ds(row0, RPT), :],
                        acc.at[pl.ds(row0, RPT), :])

        @pl.when(sid == NSUB - 1)
        def _():
            pltpu.sync_copy(base_hbm.at[pl.ds(RPT * NSUB, TAIL), :],
                            acc.at[pl.ds(RPT * NSUB, TAIL), :])

        plsc.subcore_barrier()

    def writeout(out_hbm):
        plsc.subcore_barrier()
        pltpu.sync_copy(acc.at[pl.ds(row0, RPT), :],
                        out_hbm.at[pl.ds(row0, RPT), :])

        @pl.when(sid == NSUB - 1)
        def _():
            pltpu.sync_copy(acc.at[pl.ds(RPT * NSUB, TAIL), :],
                            out_hbm.at[pl.ds(RPT * NSUB, TAIL), :])

        plsc.subcore_barrier()

    def run_pass(is_dq, xa_hbm, xb_hbm, dcol_hbm, base_hbm, out_hbm):
        init_acc(base_hbm)

        def issue_loads(s, k):
            idxj_v, _, dir_v, w12_v, ga_v, gb_v, _ = sets[s]
            e0 = ebase + k * B
            pltpu.sync_copy(idxj_hbm.at[pl.ds(e0, B)], idxj_v)
            if is_dq:
                pltpu.async_copy(w_hbm.at[pl.ds(e0, B), pl.ds(0, F)],
                                 ga_v, sem_load[s])
                pltpu.async_copy(xb_hbm.at[idxj_v], gb_v, sem_load[s])
            else:
                pltpu.async_copy(w_hbm.at[pl.ds(e0, B), pl.ds(F, 2 * F)],
                                 w12_v, sem_load[s])
                pltpu.async_copy(xa_hbm.at[idxj_v], ga_v, sem_load[s])
                pltpu.async_copy(xb_hbm.at[idxj_v], gb_v, sem_load[s])
                pltpu.sync_copy(dcol_hbm.at[pl.ds(e0, B)], dir_v)

        def wait_loads(s):
            idxj_v, _, dir_v, w12_v, ga_v, gb_v, _ = sets[s]
            if is_dq:
                pltpu.make_async_copy(w_hbm.at[pl.ds(0, B), pl.ds(0, F)],
                                      ga_v, sem_load[s]).wait()
                pltpu.make_async_copy(xb_hbm.at[idxj_v], gb_v,
                                      sem_load[s]).wait()
            else:
                pltpu.make_async_copy(w_hbm.at[pl.ds(0, B), pl.ds(F, 2 * F)],
                                      w12_v, sem_load[s]).wait()
                pltpu.make_async_copy(xa_hbm.at[idxj_v], ga_v,
                                      sem_load[s]).wait()
                pltpu.make_async_copy(xb_hbm.at[idxj_v], gb_v,
                                      sem_load[s]).wait()

        def compute(s):
            _, _, dir_v, w12_v, ga_v, gb_v, out_v = sets[s]
            if is_dq:
                @pl.loop(0, B, unroll=2)
                def _edge(e):
                    for r in range(NREG):
                        sl = pl.ds(r * 16, 16)
                        out_v[e, sl] = ga_v[e, sl] * gb_v[e, sl]
            else:
                @pl.loop(0, B, unroll=2)
                def _edge(e):
                    esplat = jnp.broadcast_to(e, (16,)).astype(jnp.int32)
                    d = plsc.load_gather(dir_v, [esplat])
                    for r in range(NREG):
                        sl = pl.ds(r * 16, 16)
                        sl2 = pl.ds(F + r * 16, 16)
                        out_v[e, sl] = (w12_v[e, sl] * ga_v[e, sl] * d
                                        + w12_v[e, sl2] * gb_v[e, sl])

        def issue_scatter(s, k):
            _, idxs_v, _, _, _, _, out_v = sets[s]
            e0 = ebase + k * B
            pltpu.sync_copy(idxi_hbm.at[pl.ds(e0, B)], idxs_v)
            pltpu.async_copy(out_v, acc.at[idxs_v], sem_scat[s], add=True)

        def wait_scatter(s):
            _, idxs_v, _, _, _, _, out_v = sets[s]
            pltpu.make_async_copy(out_v, acc.at[idxs_v], sem_scat[s]).wait()

        issue_loads(0, 0)
        issue_loads(1, 1)

        @pl.loop(0, (NCHUNK + 1) // 2)
        def _pair(m):
            for s in (0, 1):
                k = 2 * m + s

                @pl.when(k < NCHUNK)
                def _():
                    wait_loads(s)

                    @pl.when(m > 0)
                    def _():
                        wait_scatter(s)

                    compute(s)
                    issue_scatter(s, k)

                    @pl.when(k + 2 < NCHUNK)
                    def _():
                        issue_loads(s, k + 2)

        wait_scatter(0)
        wait_scatter(1)
        writeout(out_hbm)

    @pl.when(cid == 0)
    def _():
        run_pass(True, x0_hbm, x0_hbm, d0_hbm, q2_hbm, outq_hbm)
        run_pass(False, x1_hbm, p0_hbm, d0_hbm, mu0_hbm, outm0_hbm)

    @pl.when(cid == 1)
    def _():
        run_pass(False, x1_hbm, p1_hbm, d1_hbm, mu1_hbm, outm1_hbm)
        run_pass(False, x1_hbm, p2_hbm, d2_hbm, mu2_hbm, outm2_hbm)


def _set_scratch():
    return [
        pltpu.VMEM((B,), jnp.int32),
        pltpu.VMEM((B,), jnp.int32),
        pltpu.VMEM((B,), jnp.float32),
        pltpu.VMEM((B, 2 * F), jnp.float32),
        pltpu.VMEM((B, F), jnp.float32),
        pltpu.VMEM((B, F), jnp.float32),
        pltpu.VMEM((B, F), jnp.float32),
    ]


_sc_edges = functools.partial(
    pl.kernel,
    out_type=[jax.ShapeDtypeStruct((N, F), jnp.float32)] * 4,
    mesh=plsc.VectorSubcoreMesh(core_axis_name="c", subcore_axis_name="s"),
    compiler_params=pltpu.CompilerParams(needs_layout_passes=False),
    scratch_types=[pltpu.VMEM_SHARED((N, F), jnp.float32)]
    + _set_scratch() + _set_scratch()
    + [pltpu.SemaphoreType.DMA] * 4,
)(_sc_edge_kernel)


def kernel(q, mu, W_ij, dir_ij, pairlist, W1, b1, W2, b2):
    q2 = q[:, 0, :]
    mu0, mu1, mu2 = mu[:, 0, :], mu[:, 1, :], mu[:, 2, :]
    x0, x1, p0, p1, p2 = _mlp(q2, W1, b1, W2, b2, mu0, mu1, mu2)
    d0, d1, d2 = dir_ij[:, 0], dir_ij[:, 1], dir_ij[:, 2]
    idx_i, idx_j = pairlist[0], pairlist[1]
    outq, m0, m1, m2 = _sc_edges(W_ij, x0, x1, p0, p1, p2,
                                 d0, d1, d2, q2, mu0, mu1, mu2,
                                 idx_i, idx_j)
    return outq[:, None, :], jnp.stack([m0, m1, m2], axis=1)


# speedup vs baseline: 15.8925x; 1.5281x over previous
import functools

import jax
import jax.numpy as jnp
from jax import lax
from jax.experimental import pallas as pl
from jax.experimental.pallas import tpu as pltpu
from jax.experimental.pallas import tpu_sc as plsc

N = 10000
E = 320000
F = 128
NSUB = 16
B = 32
RPT = (N // NSUB) // 8 * 8
TAIL = N - RPT * NSUB
NREG = F // 16
NIDX = 8
DIDX = 6


def _mlp_body(q_ref, w1_ref, b1_ref, w2_ref, b2_ref,
              mu0_ref, mu1_ref, mu2_ref,
              x0_ref, x1_ref, p0_ref, p1_ref, p2_ref):
    h = jnp.dot(q_ref[...], w1_ref[...], preferred_element_type=jnp.float32)
    h = h + b1_ref[...]
    h = h * jax.nn.sigmoid(h)
    x = jnp.dot(h, w2_ref[...], preferred_element_type=jnp.float32) + b2_ref[...]
    x0_ref[...] = x[:, :F]
    x1_ref[...] = x[:, F:2 * F]
    x2 = x[:, 2 * F:]
    p0_ref[...] = x2 * mu0_ref[...]
    p1_ref[...] = x2 * mu1_ref[...]
    p2_ref[...] = x2 * mu2_ref[...]


def _mlp(q2, W1, b1, W2, b2, mu0, mu1, mu2):
    BN = 1000
    grid = (N // BN,)
    blk = pl.BlockSpec((BN, F), lambda i: (i, 0))
    out = pl.pallas_call(
        _mlp_body,
        grid=grid,
        in_specs=[
            blk,
            pl.BlockSpec((F, F), lambda i: (0, 0)),
            pl.BlockSpec((1, F), lambda i: (0, 0)),
            pl.BlockSpec((F, 3 * F), lambda i: (0, 0)),
            pl.BlockSpec((1, 3 * F), lambda i: (0, 0)),
            blk, blk, blk,
        ],
        out_specs=[blk] * 5,
        out_shape=[jax.ShapeDtypeStruct((N, F), jnp.float32)] * 5,
    )(q2, W1, b1.reshape(1, F), W2, b2.reshape(1, 3 * F), mu0, mu1, mu2)
    return out


def _sc_edge_kernel(w_hbm, x0_hbm, x1_hbm, pcat_hbm, dcat_hbm,
                    q2_hbm, mucat_hbm, idxi_hbm, idxj_hbm,
                    outq_hbm, outmu_hbm,
                    acc, *scr):
    cid = lax.axis_index("c")
    sid = lax.axis_index("s")
    row0 = sid * RPT
    ebase = sid * 624 * B + jnp.maximum(sid - 14, 0) * 8 * B
    nchunk = 624 + jnp.where(sid >= 14, 8, 0)

    idxj_v = scr[0:NIDX]
    idxi_v = scr[NIDX:2 * NIDX]
    idxp_v = scr[2 * NIDX:3 * NIDX]
    dir_v = scr[3 * NIDX:4 * NIDX]
    data = (scr[4 * NIDX:4 * NIDX + 4], scr[4 * NIDX + 4:4 * NIDX + 8])
    sem_idx = scr[4 * NIDX + 8:5 * NIDX + 8]
    sem_load = scr[5 * NIDX + 8:5 * NIDX + 10]
    sem_scat = scr[5 * NIDX + 10:5 * NIDX + 12]

    def init_writeout(hbm, rowoff, to_acc):
        if not to_acc:
            plsc.subcore_barrier()

        def cp(src, dst):
            pltpu.sync_copy(src, dst)

        a = acc.at[pl.ds(row0, RPT), :]
        h = hbm.at[pl.ds(rowoff + row0, RPT), :]
        cp(h, a) if to_acc else cp(a, h)

        @pl.when(sid == NSUB - 1)
        def _():
            a2 = acc.at[pl.ds(RPT * NSUB, TAIL), :]
            h2 = hbm.at[pl.ds(rowoff + RPT * NSUB, TAIL), :]
            cp(h2, a2) if to_acc else cp(a2, h2)

        plsc.subcore_barrier()

    def run_pass(is_dq, xa_hbm, xb_hbm, base_hbm, out_hbm, d):
        rowoff = d * N
        init_writeout(base_hbm, rowoff, True)

        def issue_idx(q, k):
            e0 = ebase + k * B
            pltpu.async_copy(idxj_hbm.at[pl.ds(e0, B)], idxj_v[q], sem_idx[q])
            pltpu.async_copy(idxi_hbm.at[pl.ds(e0, B)], idxi_v[q], sem_idx[q])
            if not is_dq:
                pltpu.async_copy(dcat_hbm.at[pl.ds(d * E + e0, B)], dir_v[q],
                                 sem_idx[q])

        def wait_idx(q):
            pltpu.make_async_copy(idxj_hbm.at[pl.ds(0, B)], idxj_v[q],
                                  sem_idx[q]).wait()
            pltpu.make_async_copy(idxi_hbm.at[pl.ds(0, B)], idxi_v[q],
                                  sem_idx[q]).wait()
            if not is_dq:
                pltpu.make_async_copy(dcat_hbm.at[pl.ds(0, B)], dir_v[q],
                                      sem_idx[q]).wait()

        def issue_gathers(s, q, k):
            w12_v, ga_v, gb_v, _ = data[s]
            e0 = ebase + k * B
            if is_dq:
                pltpu.async_copy(w_hbm.at[pl.ds(e0, B), pl.ds(0, F)],
                                 ga_v, sem_load[s])
                pltpu.async_copy(xb_hbm.at[idxj_v[q]], gb_v, sem_load[s])
            else:
                for t in range(B // 16):
                    sl = pl.ds(t * 16, 16)
                    idxp_v[q][sl] = idxj_v[q][sl] + rowoff
                pltpu.async_copy(w_hbm.at[pl.ds(e0, B), pl.ds(F, 2 * F)],
                                 w12_v, sem_load[s])
                pltpu.async_copy(xa_hbm.at[idxj_v[q]], ga_v, sem_load[s])
                pltpu.async_copy(xb_hbm.at[idxp_v[q]], gb_v, sem_load[s])

        def wait_loads(s, q):
            w12_v, ga_v, gb_v, _ = data[s]
            if is_dq:
                pltpu.make_async_copy(w_hbm.at[pl.ds(0, B), pl.ds(0, F)],
                                      ga_v, sem_load[s]).wait()
                pltpu.make_async_copy(xb_hbm.at[idxj_v[q]], gb_v,
                                      sem_load[s]).wait()
            else:
                pltpu.make_async_copy(w_hbm.at[pl.ds(0, B), pl.ds(F, 2 * F)],
                                      w12_v, sem_load[s]).wait()
                pltpu.make_async_copy(xa_hbm.at[idxj_v[q]], ga_v,
                                      sem_load[s]).wait()
                pltpu.make_async_copy(xb_hbm.at[idxp_v[q]], gb_v,
                                      sem_load[s]).wait()

        def compute(s, q):
            w12_v, ga_v, gb_v, out_v = data[s]
            if is_dq:
                @pl.loop(0, B, unroll=2)
                def _edge(e):
                    for r in range(NREG):
                        sl = pl.ds(r * 16, 16)
                        out_v[e, sl] = ga_v[e, sl] * gb_v[e, sl]
            else:
                dv = dir_v[q]

                @pl.loop(0, B, unroll=2)
                def _edge(e):
                    esplat = jnp.broadcast_to(e, (16,)).astype(jnp.int32)
                    dval = plsc.load_gather(dv, [esplat])
                    for r in range(NREG):
                        sl = pl.ds(r * 16, 16)
                        sl2 = pl.ds(F + r * 16, 16)
                        out_v[e, sl] = (w12_v[e, sl] * ga_v[e, sl] * dval
                                        + w12_v[e, sl2] * gb_v[e, sl])

        def issue_scatter(s, q):
            out_v = data[s][3]
            pltpu.async_copy(out_v, acc.at[idxi_v[q]], sem_scat[s], add=True)

        def wait_scatter(s, q):
            out_v = data[s][3]
            pltpu.make_async_copy(out_v, acc.at[idxi_v[q]], sem_scat[s]).wait()

        for kk in range(DIDX):
            issue_idx(kk, kk)
        wait_idx(0)
        issue_gathers(0, 0, 0)
        wait_idx(1)
        issue_gathers(1, 1, 1)

        @pl.loop(0, nchunk // NIDX)
        def _oct(mm):
            k0 = mm * NIDX
            for p in range(NIDX):
                k = k0 + p
                s = p % 2

                wait_loads(s, p)

                @pl.when(k >= 2)
                def _():
                    wait_scatter(s, (p - 2) % NIDX)

                compute(s, p)
                issue_scatter(s, p)

                @pl.when(k + DIDX < nchunk)
                def _():
                    issue_idx((p + DIDX) % NIDX, k + DIDX)

                @pl.when(k + 2 < nchunk)
                def _():
                    wait_idx((p + 2) % NIDX)
                    issue_gathers(s, (p + 2) % NIDX, k + 2)

        wait_scatter(0, (NIDX - 2) % NIDX)
        wait_scatter(1, (NIDX - 1) % NIDX)
        init_writeout(out_hbm, rowoff, False)

    @pl.when(cid == 0)
    def _():
        run_pass(True, x0_hbm, x0_hbm, q2_hbm, outq_hbm,
                 jnp.zeros((), jnp.int32))

    dlo = jnp.where(cid == 0, 0, 1)
    dhi = jnp.where(cid == 0, 1, 3)

    @pl.loop(dlo, dhi)
    def _dmu(d):
        run_pass(False, x1_hbm, pcat_hbm, mucat_hbm, outmu_hbm, d)


_sc_edges = functools.partial(
    pl.kernel,
    out_type=[jax.ShapeDtypeStruct((N, F), jnp.float32),
              jax.ShapeDtypeStruct((3 * N, F), jnp.float32)],
    mesh=plsc.VectorSubcoreMesh(core_axis_name="c", subcore_axis_name="s"),
    compiler_params=pltpu.CompilerParams(needs_layout_passes=False),
    scratch_types=[pltpu.VMEM_SHARED((N, F), jnp.float32)]
    + [pltpu.VMEM((B,), jnp.int32)] * NIDX
    + [pltpu.VMEM((B,), jnp.int32)] * NIDX
    + [pltpu.VMEM((B,), jnp.int32)] * NIDX
    + [pltpu.VMEM((B,), jnp.float32)] * NIDX
    + [pltpu.VMEM((B, 2 * F), jnp.float32),
       pltpu.VMEM((B, F), jnp.float32),
       pltpu.VMEM((B, F), jnp.float32),
       pltpu.VMEM((B, F), jnp.float32)]
    + [pltpu.VMEM((B, 2 * F), jnp.float32),
       pltpu.VMEM((B, F), jnp.float32),
       pltpu.VMEM((B, F), jnp.float32),
       pltpu.VMEM((B, F), jnp.float32)]
    + [pltpu.SemaphoreType.DMA] * NIDX
    + [pltpu.SemaphoreType.DMA] * 4,
)(_sc_edge_kernel)


def kernel(q, mu, W_ij, dir_ij, pairlist, W1, b1, W2, b2):
    q2 = q[:, 0, :]
    mu0, mu1, mu2 = mu[:, 0, :], mu[:, 1, :], mu[:, 2, :]
    x0, x1, p0, p1, p2 = _mlp(q2, W1, b1, W2, b2, mu0, mu1, mu2)
    pcat = jnp.concatenate([p0, p1, p2], axis=0)
    mucat = jnp.concatenate([mu0, mu1, mu2], axis=0)
    dcat = jnp.concatenate([dir_ij[:, 0], dir_ij[:, 1], dir_ij[:, 2]], axis=0)
    idx_i, idx_j = pairlist[0], pairlist[1]
    outq, outmu = _sc_edges(W_ij, x0, x1, pcat, dcat,
                            q2, mucat, idx_i, idx_j)
    return outq[:, None, :], outmu.reshape(3, N, F).transpose(1, 0, 2)
